# Initial kernel scaffold; baseline (speedup 1.0000x reference)
import jax
import jax.numpy as jnp
from jax.experimental import pallas as pl


def kernel(x, edge_index, W1l, b1l, W1r, W2l, b2l, W2r):
    raise NotImplementedError("write your pallas kernel here")


# speedup vs baseline: 4.3277x; 4.3277x over previous
import functools

import jax
import jax.numpy as jnp
from jax import lax
from jax.experimental import pallas as pl
from jax.experimental.pallas import tpu as pltpu
from jax.experimental.pallas import tpu_sc as plsc

_NC = 2
_NS = 16
_NW = _NC * _NS
_CH = 80
_NP = 10240
_BM = 2000


@functools.lru_cache(maxsize=None)
def _make_sc_segsum(D, E):
  epw = E // _NW
  steps = epw // _CH
  rpt = _NP // _NS
  mesh = plsc.VectorSubcoreMesh(core_axis_name="c", subcore_axis_name="s")
  out_type = jax.ShapeDtypeStruct((_NC, _NP, D), jnp.float32)
  scratch = (pltpu.VMEM_SHARED((_NP, D), jnp.float32),
             pltpu.VMEM((_CH,), jnp.int32),
             pltpu.VMEM((_CH,), jnp.int32),
             pltpu.VMEM((_CH, D), jnp.float32))

  def body(y_h, src_h, dst_h, zD_h, agg_h, acc, si, di, rows):
    c = lax.axis_index("c")
    s = lax.axis_index("s")
    wid = c * _NS + s
    r0 = s * rpt
    pltpu.sync_copy(zD_h.at[pl.ds(r0, rpt)], acc.at[pl.ds(r0, rpt)])
    plsc.subcore_barrier()
    base = wid * epw

    def step(i, carry):
      off = pl.multiple_of(base + i * _CH, 8)
      pltpu.sync_copy(src_h.at[pl.ds(off, _CH)], si)
      pltpu.sync_copy(dst_h.at[pl.ds(off, _CH)], di)
      pltpu.sync_copy(y_h.at[si], rows)
      pltpu.sync_copy(rows, acc.at[di], add=True)
      return carry

    lax.fori_loop(0, steps, step, 0)
    plsc.subcore_barrier()
    pltpu.sync_copy(acc.at[pl.ds(r0, rpt)], agg_h.at[c, pl.ds(r0, rpt)])

  return pl.kernel(body, out_type=out_type, mesh=mesh, scratch_types=scratch)


@functools.lru_cache(maxsize=None)
def _make_sc_degree(E):
  epw = E // _NW
  steps = epw // _CH
  rpt = _NP // _NS
  mesh = plsc.VectorSubcoreMesh(core_axis_name="c", subcore_axis_name="s")
  out_type = jax.ShapeDtypeStruct((_NC, _NP, 128), jnp.float32)
  scratch = (pltpu.VMEM_SHARED((_NP, 128), jnp.float32),
             pltpu.VMEM((_CH,), jnp.int32),
             pltpu.VMEM((_CH, 128), jnp.float32))

  def body(dst_h, zD_h, ones_h, cnt_h, acc, di, ones):
    c = lax.axis_index("c")
    s = lax.axis_index("s")
    wid = c * _NS + s
    r0 = s * rpt
    pltpu.sync_copy(zD_h.at[pl.ds(r0, rpt)], acc.at[pl.ds(r0, rpt)])
    pltpu.sync_copy(ones_h, ones)
    plsc.subcore_barrier()
    base = wid * epw

    def step(i, carry):
      off = pl.multiple_of(base + i * _CH, 8)
      pltpu.sync_copy(dst_h.at[pl.ds(off, _CH)], di)
      pltpu.sync_copy(ones, acc.at[di], add=True)
      return carry

    lax.fori_loop(0, steps, step, 0)
    plsc.subcore_barrier()
    pltpu.sync_copy(acc.at[pl.ds(r0, rpt)], cnt_h.at[c, pl.ds(r0, rpt)])

  return pl.kernel(body, out_type=out_type, mesh=mesh, scratch_types=scratch)


def _sc_segsum(y, src, dst):
  _, D = y.shape
  E = src.shape[0]
  k = _make_sc_segsum(D, E)
  zD = jnp.zeros((_NP, D), jnp.float32)
  return k(y, src, dst, zD)


def _sc_degree(dst):
  E = dst.shape[0]
  k = _make_sc_degree(E)
  zD = jnp.zeros((_NP, 128), jnp.float32)
  ones = jnp.ones((_CH, 128), jnp.float32)
  return k(dst, zD, ones)


def _tc_proj2(x, Wl, Wr, bl):
  N, Din = x.shape
  Dl, Dr = Wl.shape[1], Wr.shape[1]

  def body(x_ref, wl_ref, wr_ref, b_ref, y_ref, z_ref):
    xb = x_ref[...]
    y_ref[...] = jnp.dot(xb, wl_ref[...], preferred_element_type=jnp.float32)
    z_ref[...] = (jnp.dot(xb, wr_ref[...], preferred_element_type=jnp.float32)
                  + b_ref[...])

  return pl.pallas_call(
      body,
      grid=(N // _BM,),
      in_specs=[pl.BlockSpec((_BM, Din), lambda i: (i, 0)),
                pl.BlockSpec((Din, Dl), lambda i: (0, 0)),
                pl.BlockSpec((Din, Dr), lambda i: (0, 0)),
                pl.BlockSpec((1, Dr), lambda i: (0, 0))],
      out_specs=[pl.BlockSpec((_BM, Dl), lambda i: (i, 0)),
                 pl.BlockSpec((_BM, Dr), lambda i: (i, 0))],
      out_shape=[jax.ShapeDtypeStruct((N, Dl), jnp.float32),
                 jax.ShapeDtypeStruct((N, Dr), jnp.float32)],
  )(x, Wl, Wr, bl.reshape(1, Dr))


def _tc_combine_proj(aggp, cntp, z1, Wl, Wr, b):
  N, H = z1.shape
  Do = Wl.shape[1]

  def body(a_ref, c_ref, z_ref, wl_ref, wr_ref, b_ref, y_ref, z2_ref):
    a = a_ref[0] + a_ref[1]
    cnt = c_ref[0, :, 0:1] + c_ref[1, :, 0:1]
    inv = 1.0 / jnp.maximum(cnt, 1.0)
    h = jnp.maximum(a * inv + z_ref[...], 0.0)
    y_ref[...] = jnp.dot(h, wl_ref[...], preferred_element_type=jnp.float32)
    z2_ref[...] = (jnp.dot(h, wr_ref[...], preferred_element_type=jnp.float32)
                   + b_ref[...])

  return pl.pallas_call(
      body,
      grid=(N // _BM,),
      in_specs=[pl.BlockSpec((2, _BM, H), lambda i: (0, i, 0)),
                pl.BlockSpec((2, _BM, 128), lambda i: (0, i, 0)),
                pl.BlockSpec((_BM, H), lambda i: (i, 0)),
                pl.BlockSpec((H, Do), lambda i: (0, 0)),
                pl.BlockSpec((H, Do), lambda i: (0, 0)),
                pl.BlockSpec((1, Do), lambda i: (0, 0))],
      out_specs=[pl.BlockSpec((_BM, Do), lambda i: (i, 0)),
                 pl.BlockSpec((_BM, Do), lambda i: (i, 0))],
      out_shape=[jax.ShapeDtypeStruct((N, Do), jnp.float32),
                 jax.ShapeDtypeStruct((N, Do), jnp.float32)],
  )(aggp, cntp, z1, Wl, Wr, b.reshape(1, Do))


def _tc_final(aggp, cntp, z2):
  N, Do = z2.shape

  def body(a_ref, c_ref, z_ref, o_ref):
    a = a_ref[0] + a_ref[1]
    cnt = c_ref[0, :, 0:1] + c_ref[1, :, 0:1]
    inv = 1.0 / jnp.maximum(cnt, 1.0)
    o_ref[...] = a * inv + z_ref[...]

  return pl.pallas_call(
      body,
      grid=(N // _BM,),
      in_specs=[pl.BlockSpec((2, _BM, Do), lambda i: (0, i, 0)),
                pl.BlockSpec((2, _BM, 128), lambda i: (0, i, 0)),
                pl.BlockSpec((_BM, Do), lambda i: (i, 0))],
      out_specs=pl.BlockSpec((_BM, Do), lambda i: (i, 0)),
      out_shape=jax.ShapeDtypeStruct((N, Do), jnp.float32),
  )(aggp, cntp, z2)


def kernel(x, edge_index, W1l, b1l, W1r, W2l, b2l, W2r):
  N, _ = x.shape
  C = W2l.shape[1]
  Dp = 128
  src = edge_index[0].astype(jnp.int32)
  dst = edge_index[1].astype(jnp.int32)

  cntp = _sc_degree(dst)
  y1, z1 = _tc_proj2(x, W1l, W1r, b1l)
  agg1p = _sc_segsum(y1, src, dst)

  W2l_p = jnp.zeros((W2l.shape[0], Dp), jnp.float32).at[:, :C].set(W2l)
  W2r_p = jnp.zeros((W2r.shape[0], Dp), jnp.float32).at[:, :C].set(W2r)
  b2_p = jnp.zeros((Dp,), jnp.float32).at[:C].set(b2l)

  y2, z2 = _tc_combine_proj(agg1p, cntp, z1, W2l_p, W2r_p, b2_p)
  agg2p = _sc_segsum(y2, src, dst)
  out = _tc_final(agg2p, cntp, z2)
  return out[:, :C]
